# baseline (speedup 1.0000x reference)
import functools

import jax
import jax.numpy as jnp
from jax import lax
from jax.experimental import pallas as pl
from jax.experimental.pallas import tpu as pltpu
from jax.experimental.pallas import tpu_sc as plsc

B, L = 1024, 50
BL = B * L
VOCAB2 = 100002
DIM = 32
N_CONT = 4
LANES = 16
NC, NS = 2, 16
NW = NC * NS
TW = BL // NW
TC = 64
NTILES = TW // TC
MAXF = 8
MAXROWS = TC * MAXF


def _div_const(rv, d):
    if d & (d - 1) == 0:
        return lax.shift_right_logical(rv, d.bit_length() - 1)
    m = (1 << 18) // d + 1
    return lax.shift_right_logical(rv * m, 18)


def _do_group(idx_h, x_h, tab_h, w_h, b_h, out_h, F, wid,
              idxr_v, x_v, srcI_v, dstI_v, dstC_v, rows_v, cont_v,
              w_v, b_v, sem_g, sem_s):
    Fout = F + N_CONT
    TCF = TC * F
    n_disc_dma = TCF // 128
    n_cont_dma = (TC * N_CONT) // 128

    pltpu.sync_copy(w_h, w_v)
    pltpu.sync_copy(b_h, b_v)
    wv = [[w_v[i, pl.ds(16 * h, 16)] for h in range(2)] for i in range(N_CONT)]
    bv = [[b_v[i, pl.ds(16 * h, 16)] for h in range(2)] for i in range(N_CONT)]
    iota = lax.iota(jnp.int32, 16)

    def tile_body(j, carry):
        tb = wid * TW + j * TC

        pltpu.sync_copy(idx_h.at[pl.ds(tb * F, TCF)], idxr_v.at[pl.ds(0, TCF)])
        pltpu.sync_copy(x_h.at[pl.ds(tb * N_CONT, TC * N_CONT)], x_v)

        for k in range(TCF // 16):
            rv = iota + (16 * k)
            t = _div_const(rv, F)
            f = rv - t * F
            raw = idxr_v[pl.ds(16 * k, 16)]
            src = raw + (f * VOCAB2 + 1)
            dst = (t + tb) * Fout + f
            srcI_v[(16 * k) // 128, pl.ds((16 * k) % 128, 16)] = src
            dstI_v[(16 * k) // 128, pl.ds((16 * k) % 128, 16)] = dst

        gathers = []
        for c in range(n_disc_dma):
            gathers.append(pltpu.async_copy(
                tab_h.at[srcI_v.at[c]],
                rows_v.at[pl.ds(c * 128, 128)], sem_g))

        for m in range((TC * N_CONT) // 16):
            rv = iota + (16 * m)
            t = _div_const(rv, N_CONT)
            i = rv - t * N_CONT
            dst = (t + tb) * Fout + (F + i)
            dstC_v[(16 * m) // 128, pl.ds((16 * m) % 128, 16)] = dst

        def cont_body(m, c):
            xv = x_v[pl.ds(16 * m, 16)]
            for j in range(4):
                for i in range(N_CONT):
                    xs = xv[4 * j + i]
                    row = 16 * m + 4 * j + i
                    for h in range(2):
                        cont_v[row, pl.ds(16 * h, 16)] = xs * wv[i][h] + bv[i][h]
            return c
        lax.fori_loop(0, (TC * N_CONT) // 16, cont_body, 0)

        for g in gathers:
            g.wait()

        scatters = []
        for c in range(n_disc_dma):
            scatters.append(pltpu.async_copy(
                rows_v.at[pl.ds(c * 128, 128)],
                out_h.at[dstI_v.at[c]], sem_s))
        for c in range(n_cont_dma):
            scatters.append(pltpu.async_copy(
                cont_v.at[pl.ds(c * 128, 128)],
                out_h.at[dstC_v.at[c]], sem_s))
        for s in scatters:
            s.wait()
        return carry

    lax.fori_loop(0, NTILES, tile_body, 0)


def _sc_body(idx_q, idx_c, idx_f, x_q, x_c, x_f, tab_q, tab_c, tab_f,
             w_q, b_q, w_c, b_c, w_f, b_f, out_q, out_c, out_f,
             idxr_v, x_v, srcI_v, dstI_v, dstC_v, rows_v, cont_v,
             w_v, b_v, sem_g, sem_s):
    wid = lax.axis_index("s") * NC + lax.axis_index("c")
    common = (wid, idxr_v, x_v, srcI_v, dstI_v, dstC_v, rows_v, cont_v,
              w_v, b_v, sem_g, sem_s)
    _do_group(idx_q, x_q, tab_q, w_q, b_q, out_q, 6, *common)
    _do_group(idx_c, x_c, tab_c, w_c, b_c, out_c, 8, *common)
    _do_group(idx_f, x_f, tab_f, w_f, b_f, out_f, 6, *common)


_sc_kernel = pl.kernel(
    _sc_body,
    out_type=[
        jax.ShapeDtypeStruct((BL * 10, DIM), jnp.float32),
        jax.ShapeDtypeStruct((BL * 12, DIM), jnp.float32),
        jax.ShapeDtypeStruct((BL * 10, DIM), jnp.float32),
    ],
    mesh=plsc.VectorSubcoreMesh(
        core_axis_name="c", subcore_axis_name="s",
        num_cores=NC, num_subcores=NS),
    scratch_types=[
        pltpu.VMEM((MAXROWS,), jnp.int32),
        pltpu.VMEM((TC * N_CONT,), jnp.float32),
        pltpu.VMEM((MAXROWS // 128, 128), jnp.int32),
        pltpu.VMEM((MAXROWS // 128, 128), jnp.int32),
        pltpu.VMEM((TC * N_CONT // 128, 128), jnp.int32),
        pltpu.VMEM((MAXROWS, DIM), jnp.float32),
        pltpu.VMEM((TC * N_CONT, DIM), jnp.float32),
        pltpu.VMEM((N_CONT, DIM), jnp.float32),
        pltpu.VMEM((N_CONT, DIM), jnp.float32),
        pltpu.SemaphoreType.DMA,
        pltpu.SemaphoreType.DMA,
    ],
    compiler_params=pltpu.CompilerParams(use_tc_tiling_on_sc=False),
)


@jax.jit
def kernel(batch_feature_tensor_pay_QOE_discrete,
           batch_feature_tensor_pay_CHONGHE_discrete,
           batch_feature_tensor_pay_FUFEI_discrete,
           batch_feature_tensor_pay_QOE_continue,
           batch_feature_tensor_pay_CHONGHE_continue,
           batch_feature_tensor_pay_FUFEI_continue,
           QOE_tables, CHONGHE_tables, FUFEI_tables,
           W_QOE, b_QOE, W_CHONGHE, b_CHONGHE, W_FUFEI, b_FUFEI):
    idx_q = batch_feature_tensor_pay_QOE_discrete.astype(jnp.int32).reshape(BL * 6)
    idx_c = batch_feature_tensor_pay_CHONGHE_discrete.astype(jnp.int32).reshape(BL * 8)
    idx_f = batch_feature_tensor_pay_FUFEI_discrete.astype(jnp.int32).reshape(BL * 6)
    x_q = batch_feature_tensor_pay_QOE_continue.astype(jnp.float32).reshape(BL * N_CONT)
    x_c = batch_feature_tensor_pay_CHONGHE_continue.astype(jnp.float32).reshape(BL * N_CONT)
    x_f = batch_feature_tensor_pay_FUFEI_continue.astype(jnp.float32).reshape(BL * N_CONT)
    tab_q = QOE_tables.reshape(6 * VOCAB2, DIM)
    tab_c = CHONGHE_tables.reshape(8 * VOCAB2, DIM)
    tab_f = FUFEI_tables.reshape(6 * VOCAB2, DIM)
    out_q, out_c, out_f = _sc_kernel(
        idx_q, idx_c, idx_f, x_q, x_c, x_f, tab_q, tab_c, tab_f,
        W_QOE, b_QOE, W_CHONGHE, b_CHONGHE, W_FUFEI, b_FUFEI)
    return (out_q.reshape(B, L, 10, DIM),
            out_c.reshape(B, L, 12, DIM),
            out_f.reshape(B, L, 10, DIM))


# speedup vs baseline: 3.4386x; 3.4386x over previous
import jax
import jax.numpy as jnp
from jax import lax
from jax.experimental import pallas as pl
from jax.experimental.pallas import tpu as pltpu
from jax.experimental.pallas import tpu_sc as plsc

B, L = 1024, 50
V2 = 100002
DIM = 32
N_CONT = 4
NC, NS = 2, 16

_CHUNKS = [(0, 8), (8, 8), (16, 8), (24, 8), (32, 8), (40, 8), (48, 2)]


def _gather_chunk(col_v, idxc_v, stage_v, lc):
    n16 = (lc * 1024) // 16

    def body(s0, c):
        for u in range(4):
            s = s0 * 4 + u
            r = lax.shift_right_logical(s, 6)
            cc = (s & 63) * 16
            iv = idxc_v[r, pl.ds(cc, 16)]
            g = plsc.load_gather(col_v, [iv + 1])
            stage_v[r, pl.ds(cc, 16)] = g
        return c
    lax.fori_loop(0, n16 // 4, body, 0)


def _disc_group(idx_h, tab_h, out_h, F, d, ch_layout,
                col_v, idxc_v, stage_v):
    def per_feature(f, c):
        pltpu.sync_copy(tab_h.at[f, d, :], col_v)
        for l0, lc in _CHUNKS:
            if ch_layout:
                pltpu.sync_copy(idx_h.at[pl.ds(l0, lc), f, :],
                                idxc_v.at[pl.ds(0, lc), :])
            else:
                pltpu.sync_copy(idx_h.at[f, pl.ds(l0, lc), :],
                                idxc_v.at[pl.ds(0, lc), :])
            _gather_chunk(col_v, idxc_v, stage_v, lc)
            pltpu.sync_copy(stage_v.at[pl.ds(0, lc), :],
                            out_h.at[pl.ds(l0, lc), f, d, :])
        return c
    lax.fori_loop(0, F, per_feature, 0)


def _cont_group(x_h, w_h, b_h, out_h, F, d, iota,
                xc_v, stage_v, w_v, b_v):
    pltpu.sync_copy(w_h, w_v)
    pltpu.sync_copy(b_h, b_v)
    izero = iota * 0
    dv = izero + d
    wsp = [plsc.load_gather(w_v, [izero + i, dv]) for i in range(N_CONT)]
    bsp = [plsc.load_gather(b_v, [izero + i, dv]) for i in range(N_CONT)]

    def per_pair(lp, c):
        l0 = lp * 2
        pltpu.sync_copy(x_h.at[pl.ds(l0 * 4, 8), :], xc_v)

        def body(s, cc):
            for i in range(N_CONT):
                for dl in range(2):
                    xv = xc_v[dl * N_CONT + i, pl.ds(s * 16, 16)]
                    stage_v[i * 2 + dl, pl.ds(s * 16, 16)] = xv * wsp[i] + bsp[i]
            return cc
        lax.fori_loop(0, 64, body, 0)
        for i in range(N_CONT):
            pltpu.sync_copy(stage_v.at[pl.ds(i * 2, 2), :],
                            out_h.at[pl.ds(l0, 2), F + i, d, :])
        return c
    lax.fori_loop(0, L // 2, per_pair, 0)


def _sc_body(idx_q, idx_c, idx_f, x_q, x_c, x_f, tab_q, tab_c, tab_f,
             w_q, b_q, w_c, b_c, w_f, b_f, out_q, out_c, out_f,
             col_v, idxc_v, stage_v, w_v, b_v):
    d = lax.axis_index("s") * NC + lax.axis_index("c")
    iota = lax.iota(jnp.int32, 16)
    xc_v = idxc_v.bitcast(jnp.float32)

    _disc_group(idx_q, tab_q, out_q, 6, d, False, col_v, idxc_v, stage_v)
    _disc_group(idx_c, tab_c, out_c, 8, d, True, col_v, idxc_v, stage_v)
    _disc_group(idx_f, tab_f, out_f, 6, d, False, col_v, idxc_v, stage_v)

    _cont_group(x_q, w_q, b_q, out_q, 6, d, iota, xc_v, stage_v, w_v, b_v)
    _cont_group(x_c, w_c, b_c, out_c, 8, d, iota, xc_v, stage_v, w_v, b_v)
    _cont_group(x_f, w_f, b_f, out_f, 6, d, iota, xc_v, stage_v, w_v, b_v)


_sc_kernel = pl.kernel(
    _sc_body,
    out_type=[
        jax.ShapeDtypeStruct((L, 10, DIM, B), jnp.float32),
        jax.ShapeDtypeStruct((L, 12, DIM, B), jnp.float32),
        jax.ShapeDtypeStruct((L, 10, DIM, B), jnp.float32),
    ],
    mesh=plsc.VectorSubcoreMesh(
        core_axis_name="c", subcore_axis_name="s",
        num_cores=NC, num_subcores=NS),
    scratch_types=[
        pltpu.VMEM((V2,), jnp.float32),
        pltpu.VMEM((8, B), jnp.int32),
        pltpu.VMEM((8, B), jnp.float32),
        pltpu.VMEM((N_CONT, DIM), jnp.float32),
        pltpu.VMEM((N_CONT, DIM), jnp.float32),
    ],
    compiler_params=pltpu.CompilerParams(needs_layout_passes=False),
)


@jax.jit
def kernel(batch_feature_tensor_pay_QOE_discrete,
           batch_feature_tensor_pay_CHONGHE_discrete,
           batch_feature_tensor_pay_FUFEI_discrete,
           batch_feature_tensor_pay_QOE_continue,
           batch_feature_tensor_pay_CHONGHE_continue,
           batch_feature_tensor_pay_FUFEI_continue,
           QOE_tables, CHONGHE_tables, FUFEI_tables,
           W_QOE, b_QOE, W_CHONGHE, b_CHONGHE, W_FUFEI, b_FUFEI):
    idx_q = batch_feature_tensor_pay_QOE_discrete.astype(jnp.int32).transpose(2, 1, 0)
    idx_c = batch_feature_tensor_pay_CHONGHE_discrete.astype(jnp.int32).transpose(1, 2, 0)
    idx_f = batch_feature_tensor_pay_FUFEI_discrete.astype(jnp.int32).transpose(2, 1, 0)
    x_q = batch_feature_tensor_pay_QOE_continue.astype(jnp.float32).transpose(1, 2, 0).reshape(L * N_CONT, B)
    x_c = batch_feature_tensor_pay_CHONGHE_continue.astype(jnp.float32).transpose(1, 2, 0).reshape(L * N_CONT, B)
    x_f = batch_feature_tensor_pay_FUFEI_continue.astype(jnp.float32).transpose(1, 2, 0).reshape(L * N_CONT, B)
    tab_q = QOE_tables.transpose(0, 2, 1)
    tab_c = CHONGHE_tables.transpose(0, 2, 1)
    tab_f = FUFEI_tables.transpose(0, 2, 1)
    out_q, out_c, out_f = _sc_kernel(
        idx_q, idx_c, idx_f, x_q, x_c, x_f, tab_q, tab_c, tab_f,
        W_QOE, b_QOE, W_CHONGHE, b_CHONGHE, W_FUFEI, b_FUFEI)
    return (out_q.transpose(3, 0, 1, 2),
            out_c.transpose(3, 0, 1, 2),
            out_f.transpose(3, 0, 1, 2))


# speedup vs baseline: 5.0470x; 1.4678x over previous
import jax
import jax.numpy as jnp
from jax import lax
from jax.experimental import pallas as pl
from jax.experimental.pallas import tpu as pltpu
from jax.experimental.pallas import tpu_sc as plsc

B, L = 1024, 50
V2 = 100002
DIM = 32
N_CONT = 4
NC, NS = 2, 16
NPAIR = L // 2


def _div_const(rv, d):
    if d & (d - 1) == 0:
        return lax.shift_right_logical(rv, d.bit_length() - 1)
    m = (1 << 18) // d + 1
    return lax.shift_right_logical(rv * m, 18)


def _gather_chunk(col_v, idxc_v, stage_v, lc):
    @plsc.parallel_loop(0, (lc * 1024) // 16, unroll=8)
    def _(s):
        r = lax.shift_right_logical(s, 6)
        cc = (s & 63) * 16
        iv = idxc_v[r, pl.ds(cc, 16)]
        stage_v[r, pl.ds(cc, 16)] = plsc.load_gather(col_v, [iv + 1])


def _cont_pair(x_h, out_h, F, d, lp, wsp, bsp, xc_v, stage_v):
    l0 = lp * 2
    pltpu.sync_copy(x_h.at[pl.ds(l0 * N_CONT, 8), :], xc_v)

    def body(s, c):
        for r in range(8):
            i = r % N_CONT
            dl = r // N_CONT
            xv = xc_v[r, pl.ds(s * 16, 16)]
            stage_v[i * 2 + dl, pl.ds(s * 16, 16)] = xv * wsp[i] + bsp[i]
        return c
    lax.fori_loop(0, B // 16, body, 0)
    for i in range(N_CONT):
        pltpu.sync_copy(stage_v.at[pl.ds(i * 2, 2), :],
                        out_h.at[pl.ds(l0, 2), F + i, d, :])


def _group(idx_h, tab_h, x_h, w_h, b_h, out_h, F, d, ch_layout, iota,
           col_v, idxc_v, stage_a, stage_b, w_v, b_v, sem_c, sem_o):
    pltpu.sync_copy(w_h, w_v)
    pltpu.sync_copy(b_h, b_v)
    izero = iota * 0
    dv = izero + d
    wsp = [plsc.load_gather(w_v, [izero + i, dv]) for i in range(N_CONT)]
    bsp = [plsc.load_gather(b_v, [izero + i, dv]) for i in range(N_CONT)]
    xc_v = idxc_v.bitcast(jnp.float32)

    def per_feature(f, c):
        hcol = pltpu.async_copy(tab_h.at[f, d, :], col_v, sem_c)

        lo = _div_const(f * NPAIR, F)
        hi = _div_const((f + 1) * NPAIR, F)

        def pair_body(lp, cc):
            _cont_pair(x_h, out_h, F, d, lp, wsp, bsp, xc_v, stage_a)
            return cc
        lax.fori_loop(lo, hi, pair_body, 0)

        hcol.wait()

        stages = (stage_a, stage_b)
        houts = [None, None]
        for cj in range(6):
            stg = stages[cj % 2]
            if houts[cj % 2] is not None:
                houts[cj % 2].wait()
            if ch_layout:
                pltpu.sync_copy(idx_h.at[pl.ds(cj * 8, 8), f, :], idxc_v)
            else:
                pltpu.sync_copy(idx_h.at[f, pl.ds(cj * 8, 8), :], idxc_v)
            _gather_chunk(col_v, idxc_v, stg, 8)
            houts[cj % 2] = pltpu.async_copy(
                stg, out_h.at[pl.ds(cj * 8, 8), f, d, :], sem_o)
        houts[0].wait()
        houts[1].wait()
        if ch_layout:
            pltpu.sync_copy(idx_h.at[pl.ds(48, 2), f, :],
                            idxc_v.at[pl.ds(0, 2), :])
        else:
            pltpu.sync_copy(idx_h.at[f, pl.ds(48, 2), :],
                            idxc_v.at[pl.ds(0, 2), :])
        _gather_chunk(col_v, idxc_v, stage_a, 2)
        pltpu.sync_copy(stage_a.at[pl.ds(0, 2), :],
                        out_h.at[pl.ds(48, 2), f, d, :])
        return c
    lax.fori_loop(0, F, per_feature, 0)


def _sc_body(idx_q, idx_c, idx_f, x_q, x_c, x_f, tab_q, tab_c, tab_f,
             w_q, b_q, w_c, b_c, w_f, b_f, out_q, out_c, out_f,
             col_v, idxc_v, stage_a, stage_b, w_v, b_v, sem_c, sem_o):
    d = lax.axis_index("s") * NC + lax.axis_index("c")
    iota = lax.iota(jnp.int32, 16)
    common = (d,)
    _group(idx_q, tab_q, x_q, w_q, b_q, out_q, 6, d, False, iota,
           col_v, idxc_v, stage_a, stage_b, w_v, b_v, sem_c, sem_o)
    _group(idx_c, tab_c, x_c, w_c, b_c, out_c, 8, d, True, iota,
           col_v, idxc_v, stage_a, stage_b, w_v, b_v, sem_c, sem_o)
    _group(idx_f, tab_f, x_f, w_f, b_f, out_f, 6, d, False, iota,
           col_v, idxc_v, stage_a, stage_b, w_v, b_v, sem_c, sem_o)


_sc_kernel = pl.kernel(
    _sc_body,
    out_type=[
        jax.ShapeDtypeStruct((L, 10, DIM, B), jnp.float32),
        jax.ShapeDtypeStruct((L, 12, DIM, B), jnp.float32),
        jax.ShapeDtypeStruct((L, 10, DIM, B), jnp.float32),
    ],
    mesh=plsc.VectorSubcoreMesh(
        core_axis_name="c", subcore_axis_name="s",
        num_cores=NC, num_subcores=NS),
    scratch_types=[
        pltpu.VMEM((V2,), jnp.float32),
        pltpu.VMEM((8, B), jnp.int32),
        pltpu.VMEM((8, B), jnp.float32),
        pltpu.VMEM((8, B), jnp.float32),
        pltpu.VMEM((N_CONT, DIM), jnp.float32),
        pltpu.VMEM((N_CONT, DIM), jnp.float32),
        pltpu.SemaphoreType.DMA,
        pltpu.SemaphoreType.DMA,
    ],
    compiler_params=pltpu.CompilerParams(needs_layout_passes=False),
)


@jax.jit
def kernel(batch_feature_tensor_pay_QOE_discrete,
           batch_feature_tensor_pay_CHONGHE_discrete,
           batch_feature_tensor_pay_FUFEI_discrete,
           batch_feature_tensor_pay_QOE_continue,
           batch_feature_tensor_pay_CHONGHE_continue,
           batch_feature_tensor_pay_FUFEI_continue,
           QOE_tables, CHONGHE_tables, FUFEI_tables,
           W_QOE, b_QOE, W_CHONGHE, b_CHONGHE, W_FUFEI, b_FUFEI):
    idx_q = batch_feature_tensor_pay_QOE_discrete.astype(jnp.int32).transpose(2, 1, 0)
    idx_c = batch_feature_tensor_pay_CHONGHE_discrete.astype(jnp.int32).transpose(1, 2, 0)
    idx_f = batch_feature_tensor_pay_FUFEI_discrete.astype(jnp.int32).transpose(2, 1, 0)
    x_q = batch_feature_tensor_pay_QOE_continue.astype(jnp.float32).transpose(1, 2, 0).reshape(L * N_CONT, B)
    x_c = batch_feature_tensor_pay_CHONGHE_continue.astype(jnp.float32).transpose(1, 2, 0).reshape(L * N_CONT, B)
    x_f = batch_feature_tensor_pay_FUFEI_continue.astype(jnp.float32).transpose(1, 2, 0).reshape(L * N_CONT, B)
    tab_q = QOE_tables.transpose(0, 2, 1)
    tab_c = CHONGHE_tables.transpose(0, 2, 1)
    tab_f = FUFEI_tables.transpose(0, 2, 1)
    out_q, out_c, out_f = _sc_kernel(
        idx_q, idx_c, idx_f, x_q, x_c, x_f, tab_q, tab_c, tab_f,
        W_QOE, b_QOE, W_CHONGHE, b_CHONGHE, W_FUFEI, b_FUFEI)
    return (out_q.transpose(3, 0, 1, 2),
            out_c.transpose(3, 0, 1, 2),
            out_f.transpose(3, 0, 1, 2))


# speedup vs baseline: 5.1965x; 1.0296x over previous
import jax
import jax.numpy as jnp
from jax import lax
from jax.experimental import pallas as pl
from jax.experimental.pallas import tpu as pltpu
from jax.experimental.pallas import tpu_sc as plsc

B, L = 1024, 50
V2 = 100002
DIM = 32
N_CONT = 4
NC, NS = 2, 16
NPAIR = L // 2


def _div_const(rv, d):
    if d & (d - 1) == 0:
        return lax.shift_right_logical(rv, d.bit_length() - 1)
    m = (1 << 18) // d + 1
    return lax.shift_right_logical(rv * m, 18)


def _gather_chunk(col_v, idxc_v, stage_v, lc):
    @plsc.parallel_loop(0, (lc * 1024) // 16, unroll=8)
    def _(s):
        r = lax.shift_right_logical(s, 6)
        cc = (s & 63) * 16
        iv = idxc_v[r, pl.ds(cc, 16)]
        stage_v[r, pl.ds(cc, 16)] = plsc.load_gather(col_v, [iv + 1])


def _cont_unit(x_h, out_h, F, d, u, w4_v, b4_v, xc_v, stage_v, sem_o):
    i = _div_const(u, 6)
    o = u - i * 6
    pltpu.sync_copy(x_h.at[i, pl.ds(o * 8, 8), :], xc_v)
    wv = w4_v[i, pl.ds(0, 16)]
    bv = b4_v[i, pl.ds(0, 16)]

    def body(s, c):
        for dl in range(8):
            xv = xc_v[dl, pl.ds(s * 16, 16)]
            stage_v[dl, pl.ds(s * 16, 16)] = xv * wv + bv
        return c
    lax.fori_loop(0, B // 16, body, 0)
    return pltpu.async_copy(
        stage_v, out_h.at[pl.ds(o * 8, 8), F + i, d, :], sem_o)


def _group(idx_h, tab_h, x_h, w_h, b_h, out_h, F, d, ch_layout, iota,
           col_v, idxc_v, stage_a, stage_b, w_v, b_v, w4_v, b4_v,
           sem_c, sem_o):
    pltpu.sync_copy(w_h, w_v)
    pltpu.sync_copy(b_h, b_v)
    izero = iota * 0
    dv = izero + d
    for i in range(N_CONT):
        w4_v[i, pl.ds(0, 16)] = plsc.load_gather(w_v, [izero + i, dv])
        b4_v[i, pl.ds(0, 16)] = plsc.load_gather(b_v, [izero + i, dv])
    xc_v = idxc_v.bitcast(jnp.float32)
    upf = 24 // F

    def per_feature(f, c):
        hcol = pltpu.async_copy(tab_h.at[f, d, :], col_v, sem_c)

        stages = (stage_a, stage_b)
        houts = [None, None]
        slot = 0
        for k in range(upf):
            stg = stages[slot % 2]
            _ = houts[slot % 2].wait() if houts[slot % 2] is not None else None
            houts[slot % 2] = _cont_unit(
                x_h, out_h, F, d, f * upf + k, w4_v, b4_v, xc_v, stg, sem_o)
            slot += 1

        hcol.wait()

        for cj in range(6):
            stg = stages[slot % 2]
            if houts[slot % 2] is not None:
                houts[slot % 2].wait()
            if ch_layout:
                pltpu.sync_copy(idx_h.at[pl.ds(cj * 8, 8), f, :], idxc_v)
            else:
                pltpu.sync_copy(idx_h.at[f, pl.ds(cj * 8, 8), :], idxc_v)
            _gather_chunk(col_v, idxc_v, stg, 8)
            houts[slot % 2] = pltpu.async_copy(
                stg, out_h.at[pl.ds(cj * 8, 8), f, d, :], sem_o)
            slot += 1
        houts[0].wait()
        houts[1].wait()
        if ch_layout:
            pltpu.sync_copy(idx_h.at[pl.ds(48, 2), f, :],
                            idxc_v.at[pl.ds(0, 2), :])
        else:
            pltpu.sync_copy(idx_h.at[f, pl.ds(48, 2), :],
                            idxc_v.at[pl.ds(0, 2), :])
        _gather_chunk(col_v, idxc_v, stage_a, 2)
        pltpu.sync_copy(stage_a.at[pl.ds(0, 2), :],
                        out_h.at[pl.ds(48, 2), f, d, :])
        return c
    lax.fori_loop(0, F, per_feature, 0)

    for i in range(N_CONT):
        pltpu.sync_copy(x_h.at[i, pl.ds(48, 2), :],
                        xc_v.at[pl.ds(0, 2), :])
        wv = w4_v[i, pl.ds(0, 16)]
        bv = b4_v[i, pl.ds(0, 16)]

        def tail_body(s, c, i=i, wv=wv, bv=bv):
            for dl in range(2):
                xv = xc_v[dl, pl.ds(s * 16, 16)]
                stage_a[dl, pl.ds(s * 16, 16)] = xv * wv + bv
            return c
        lax.fori_loop(0, B // 16, tail_body, 0)
        pltpu.sync_copy(stage_a.at[pl.ds(0, 2), :],
                        out_h.at[pl.ds(48, 2), F + i, d, :])


def _sc_body(idx_q, idx_c, idx_f, x_q, x_c, x_f, tab_q, tab_c, tab_f,
             w_q, b_q, w_c, b_c, w_f, b_f, out_q, out_c, out_f,
             col_v, idxc_v, stage_a, stage_b, w_v, b_v, w4_v, b4_v,
             sem_c, sem_o):
    d = lax.axis_index("s") * NC + lax.axis_index("c")
    iota = lax.iota(jnp.int32, 16)
    rest = (col_v, idxc_v, stage_a, stage_b, w_v, b_v, w4_v, b4_v,
            sem_c, sem_o)
    _group(idx_q, tab_q, x_q, w_q, b_q, out_q, 6, d, False, iota, *rest)
    _group(idx_c, tab_c, x_c, w_c, b_c, out_c, 8, d, True, iota, *rest)
    _group(idx_f, tab_f, x_f, w_f, b_f, out_f, 6, d, False, iota, *rest)


_sc_kernel = pl.kernel(
    _sc_body,
    out_type=[
        jax.ShapeDtypeStruct((L, 10, DIM, B), jnp.float32),
        jax.ShapeDtypeStruct((L, 12, DIM, B), jnp.float32),
        jax.ShapeDtypeStruct((L, 10, DIM, B), jnp.float32),
    ],
    mesh=plsc.VectorSubcoreMesh(
        core_axis_name="c", subcore_axis_name="s",
        num_cores=NC, num_subcores=NS),
    scratch_types=[
        pltpu.VMEM((V2,), jnp.float32),
        pltpu.VMEM((8, B), jnp.int32),
        pltpu.VMEM((8, B), jnp.float32),
        pltpu.VMEM((8, B), jnp.float32),
        pltpu.VMEM((N_CONT, DIM), jnp.float32),
        pltpu.VMEM((N_CONT, DIM), jnp.float32),
        pltpu.VMEM((N_CONT, 16), jnp.float32),
        pltpu.VMEM((N_CONT, 16), jnp.float32),
        pltpu.SemaphoreType.DMA,
        pltpu.SemaphoreType.DMA,
    ],
    compiler_params=pltpu.CompilerParams(needs_layout_passes=False),
)


@jax.jit
def kernel(batch_feature_tensor_pay_QOE_discrete,
           batch_feature_tensor_pay_CHONGHE_discrete,
           batch_feature_tensor_pay_FUFEI_discrete,
           batch_feature_tensor_pay_QOE_continue,
           batch_feature_tensor_pay_CHONGHE_continue,
           batch_feature_tensor_pay_FUFEI_continue,
           QOE_tables, CHONGHE_tables, FUFEI_tables,
           W_QOE, b_QOE, W_CHONGHE, b_CHONGHE, W_FUFEI, b_FUFEI):
    idx_q = batch_feature_tensor_pay_QOE_discrete.astype(jnp.int32).transpose(2, 1, 0)
    idx_c = batch_feature_tensor_pay_CHONGHE_discrete.astype(jnp.int32).transpose(1, 2, 0)
    idx_f = batch_feature_tensor_pay_FUFEI_discrete.astype(jnp.int32).transpose(2, 1, 0)
    x_q = batch_feature_tensor_pay_QOE_continue.astype(jnp.float32).transpose(2, 1, 0)
    x_c = batch_feature_tensor_pay_CHONGHE_continue.astype(jnp.float32).transpose(2, 1, 0)
    x_f = batch_feature_tensor_pay_FUFEI_continue.astype(jnp.float32).transpose(2, 1, 0)
    tab_q = QOE_tables.transpose(0, 2, 1)
    tab_c = CHONGHE_tables.transpose(0, 2, 1)
    tab_f = FUFEI_tables.transpose(0, 2, 1)
    out_q, out_c, out_f = _sc_kernel(
        idx_q, idx_c, idx_f, x_q, x_c, x_f, tab_q, tab_c, tab_f,
        W_QOE, b_QOE, W_CHONGHE, b_CHONGHE, W_FUFEI, b_FUFEI)
    return (out_q.transpose(3, 0, 1, 2),
            out_c.transpose(3, 0, 1, 2),
            out_f.transpose(3, 0, 1, 2))
